# baseline (speedup 1.0000x reference)
import functools

import jax
import jax.numpy as jnp
from jax import lax
from jax.experimental import pallas as pl
from jax.experimental.pallas import tpu as pltpu
from jax.experimental.pallas import tpu_sc as plsc

D = 768
EPS = 1e-12
_NC, _NS = 2, 16
_NW = _NC * _NS
_CH = 64


def _sc_gather(word_table, idx3):
    nw, n_ch, ch = idx3.shape
    b_total = nw * n_ch * ch
    mesh = plsc.VectorSubcoreMesh(core_axis_name="c", subcore_axis_name="s")

    @functools.partial(
        pl.kernel,
        mesh=mesh,
        out_type=jax.ShapeDtypeStruct((b_total, D), jnp.float32),
        scratch_types=[
            pltpu.VMEM((n_ch, ch), jnp.int32),
            pltpu.VMEM((ch, D), jnp.float32),
            pltpu.VMEM((ch, D), jnp.float32),
            pltpu.SemaphoreType.DMA,
            pltpu.SemaphoreType.DMA,
            pltpu.SemaphoreType.DMA,
            pltpu.SemaphoreType.DMA,
        ],
    )
    def k(table_hbm, idx_hbm, out_hbm, idx_v, buf0, buf1, g0, g1, o0, o1):
        wid = lax.axis_index("s") * _NC + lax.axis_index("c")
        base = wid * (n_ch * ch)
        pltpu.sync_copy(idx_hbm.at[wid], idx_v)
        bufs = (buf0, buf1)
        gsems = (g0, g1)
        osems = (o0, o1)
        gh = [None] * n_ch
        oh = [None] * n_ch
        gh[0] = pltpu.async_copy(table_hbm.at[idx_v.at[0]], bufs[0], gsems[0])
        if n_ch > 1:
            gh[1] = pltpu.async_copy(table_hbm.at[idx_v.at[1]], bufs[1], gsems[1])
        for c in range(n_ch):
            b = c % 2
            gh[c].wait()
            oh[c] = pltpu.async_copy(
                bufs[b], out_hbm.at[pl.ds(base + c * ch, ch)], osems[b])
            if c + 2 < n_ch:
                oh[c].wait()
                gh[c + 2] = pltpu.async_copy(
                    table_hbm.at[idx_v.at[c + 2]], bufs[b], gsems[b])
        if n_ch >= 2:
            oh[n_ch - 2].wait()
        oh[n_ch - 1].wait()

    return k(word_table, idx3)


def _ln_body(tt_ref, tab_ref, gamma_ref, x_ref, o_ref):
    x = x_ref[...]
    ttf = tt_ref[0]
    t0 = tab_ref[0, :][None, :]
    dt = tab_ref[1, :][None, :] - t0
    x = x + t0 + ttf * dt
    mean = jnp.mean(x, axis=1, keepdims=True)
    xc = x - mean
    var = jnp.mean(xc * xc, axis=1, keepdims=True)
    o_ref[...] = xc * lax.rsqrt(var + EPS) * gamma_ref[0, :][None, :]


def _tc_layernorm(gathered, token_type_flat, type_table, gamma, tb=2048):
    b_total = gathered.shape[0]
    nb = b_total // tb
    tt3 = token_type_flat.reshape(nb, tb, 1).astype(jnp.float32)
    gamma2 = gamma.reshape(1, D)
    return pl.pallas_call(
        _ln_body,
        grid=(nb,),
        in_specs=[
            pl.BlockSpec((1, tb, 1), lambda i: (i, 0, 0)),
            pl.BlockSpec((2, D), lambda i: (0, 0)),
            pl.BlockSpec((1, D), lambda i: (0, 0)),
            pl.BlockSpec((tb, D), lambda i: (i, 0)),
        ],
        out_specs=pl.BlockSpec((tb, D), lambda i: (i, 0)),
        out_shape=jax.ShapeDtypeStruct((b_total, D), jnp.float32),
    )(tt3, type_table, gamma2, gathered)


def kernel(input_ids, token_type_ids, word_table, type_table, gamma):
    batch, seq = input_ids.shape
    b_total = batch * seq
    k_chunks = 4
    cb = b_total // k_chunks
    n_ch = cb // (_NW * _CH)
    idx4 = input_ids.reshape(k_chunks, _NW, n_ch, _CH)
    tt2 = token_type_ids.reshape(k_chunks, cb)
    outs = []
    for k in range(k_chunks):
        gathered = _sc_gather(word_table, idx4[k])
        outs.append(_tc_layernorm(gathered, tt2[k], type_table, gamma))
    out = jnp.concatenate(outs, axis=0)
    return out.reshape(batch, seq, D)


# speedup vs baseline: 1.5036x; 1.5036x over previous
import functools

import jax
import jax.numpy as jnp
from jax import lax
from jax.experimental import pallas as pl
from jax.experimental.pallas import tpu as pltpu
from jax.experimental.pallas import tpu_sc as plsc

D = 768
EPS = 1e-12
_NC, _NS = 2, 16
_NW = _NC * _NS
_CH = 32
_NBUF = 4


def _sc_gather(word_table, idx3):
    nw, n_ch, ch = idx3.shape
    b_total = nw * n_ch * ch
    mesh = plsc.VectorSubcoreMesh(core_axis_name="c", subcore_axis_name="s")

    @functools.partial(
        pl.kernel,
        mesh=mesh,
        out_type=jax.ShapeDtypeStruct((b_total, D), jnp.float32),
        scratch_types=[
            pltpu.VMEM((n_ch, ch), jnp.int32),
        ] + [pltpu.VMEM((ch, D), jnp.float32)] * _NBUF
          + [pltpu.SemaphoreType.DMA] * (2 * _NBUF),
    )
    def k(table_hbm, idx_hbm, out_hbm, idx_v, *bufs_sems):
        bufs = bufs_sems[:_NBUF]
        gsems = bufs_sems[_NBUF:2 * _NBUF]
        osems = bufs_sems[2 * _NBUF:]
        wid = lax.axis_index("s") * _NC + lax.axis_index("c")
        base = wid * (n_ch * ch)
        pltpu.sync_copy(idx_hbm.at[wid], idx_v)
        gh = [None] * n_ch
        oh = [None] * n_ch
        for c in range(min(_NBUF - 1, n_ch)):
            gh[c] = pltpu.async_copy(
                table_hbm.at[idx_v.at[c]], bufs[c], gsems[c])
        for c in range(n_ch):
            b = c % _NBUF
            gh[c].wait()
            oh[c] = pltpu.async_copy(
                bufs[b], out_hbm.at[pl.ds(base + c * ch, ch)], osems[b])
            if c >= 1:
                oh[c - 1].wait()
            nxt = c + _NBUF - 1
            if nxt < n_ch:
                nb = nxt % _NBUF
                gh[nxt] = pltpu.async_copy(
                    table_hbm.at[idx_v.at[nxt]], bufs[nb], gsems[nb])
        oh[n_ch - 1].wait()

    return k(word_table, idx3)


def _ln_body(tt_ref, tab_ref, gamma_ref, x_ref, o_ref):
    x = x_ref[...]
    ttf = tt_ref[0]
    t0 = tab_ref[0, :][None, :]
    dt = tab_ref[1, :][None, :] - t0
    x = x + t0 + ttf * dt
    mean = jnp.mean(x, axis=1, keepdims=True)
    xc = x - mean
    var = jnp.mean(xc * xc, axis=1, keepdims=True)
    o_ref[...] = xc * lax.rsqrt(var + EPS) * gamma_ref[0, :][None, :]


def _tc_layernorm(gathered, token_type_flat, type_table, gamma, tb=2048):
    b_total = gathered.shape[0]
    nb = b_total // tb
    tt3 = token_type_flat.reshape(nb, tb, 1).astype(jnp.float32)
    gamma2 = gamma.reshape(1, D)
    return pl.pallas_call(
        _ln_body,
        grid=(nb,),
        in_specs=[
            pl.BlockSpec((1, tb, 1), lambda i: (i, 0, 0)),
            pl.BlockSpec((2, D), lambda i: (0, 0)),
            pl.BlockSpec((1, D), lambda i: (0, 0)),
            pl.BlockSpec((tb, D), lambda i: (i, 0)),
        ],
        out_specs=pl.BlockSpec((tb, D), lambda i: (i, 0)),
        out_shape=jax.ShapeDtypeStruct((b_total, D), jnp.float32),
    )(tt3, type_table, gamma2, gathered)


def kernel(input_ids, token_type_ids, word_table, type_table, gamma):
    batch, seq = input_ids.shape
    b_total = batch * seq
    n_ch = b_total // (_NW * _CH)
    idx3 = input_ids.reshape(_NW, n_ch, _CH)
    gathered = _sc_gather(word_table, idx3)
    out = _tc_layernorm(gathered, token_type_ids.reshape(-1), type_table, gamma)
    return out.reshape(batch, seq, D)
